# Initial kernel scaffold; baseline (speedup 1.0000x reference)
import functools

import jax
import jax.numpy as jnp
from jax import lax
from jax.experimental import pallas as pl
from jax.experimental.pallas import tpu as pltpu
from jax.experimental.pallas import tpu_sc as plsc

B = 16384
D = 64

NC = 2
NS = 16
NW = NC * NS

IDX_CHUNK = 128
BAG_CH = (B * 10) // (NW * IDX_CHUNK)
EMB_CH = (B * 3) // (NW * IDX_CHUNK)
GPO = 4


def _sc_gather_body(bag_idx, emb_idx, bag_table, emb_table,
                    out_bag, out_emb, idx_b_v, idx_e_v, rows_v, sem):
  wid = lax.axis_index("s") * NC + lax.axis_index("c")
  pltpu.sync_copy(bag_idx.at[wid], idx_b_v)
  pltpu.sync_copy(emb_idx.at[wid], idx_e_v)

  def bag_body(it, carry):
    cps = [
        pltpu.async_copy(bag_table.at[idx_b_v.at[it * GPO + g]],
                         rows_v.at[pl.ds(g * IDX_CHUNK, IDX_CHUNK)], sem)
        for g in range(GPO)
    ]
    for c in cps:
      c.wait()
    base = wid * (BAG_CH * IDX_CHUNK) + it * (GPO * IDX_CHUNK)
    pltpu.sync_copy(rows_v, out_bag.at[pl.ds(base, GPO * IDX_CHUNK)])
    return carry

  lax.fori_loop(0, BAG_CH // GPO, bag_body, 0, unroll=False)

  def emb_body(it, carry):
    cps = [
        pltpu.async_copy(emb_table.at[idx_e_v.at[it * GPO + g]],
                         rows_v.at[pl.ds(g * IDX_CHUNK, IDX_CHUNK)], sem)
        for g in range(GPO)
    ]
    for c in cps:
      c.wait()
    base = wid * (EMB_CH * IDX_CHUNK) + it * (GPO * IDX_CHUNK)
    pltpu.sync_copy(rows_v, out_emb.at[pl.ds(base, GPO * IDX_CHUNK)])
    return carry

  lax.fori_loop(0, EMB_CH // GPO, emb_body, 0, unroll=False)


def _sc_gather(bag_idx, emb_idx, bag_table, emb_table):
  mesh = plsc.VectorSubcoreMesh(core_axis_name="c", subcore_axis_name="s")
  fn = pl.kernel(
      _sc_gather_body,
      out_type=[
          jax.ShapeDtypeStruct((B * 10, D), jnp.float32),
          jax.ShapeDtypeStruct((B * 3, D), jnp.float32),
      ],
      mesh=mesh,
      scratch_types=[
          pltpu.VMEM((BAG_CH, IDX_CHUNK), jnp.int32),
          pltpu.VMEM((EMB_CH, IDX_CHUNK), jnp.int32),
          pltpu.VMEM((GPO * IDX_CHUNK, D), jnp.float32),
          pltpu.SemaphoreType.DMA,
      ],
  )
  return fn(bag_idx, emb_idx, bag_table, emb_table)


R = 1024


def _mlp_body(g1, g2, info, w1a, w1b, w1c, b1, w2, b2, w3, b3, out):
  h = (
      jnp.dot(g1[...], w1a[...], preferred_element_type=jnp.float32)
      + jnp.dot(g2[...], w1b[...], preferred_element_type=jnp.float32)
      + jnp.dot(info[...], w1c[...], preferred_element_type=jnp.float32)
      + b1[...]
  )
  h = jnp.maximum(h, 0.0)
  h = jnp.maximum(jnp.dot(h, w2[...], preferred_element_type=jnp.float32)
                  + b2[...], 0.0)
  out[...] = jnp.dot(h, w3[...], preferred_element_type=jnp.float32) + b3[...]


def _mlp(g1, g2, info, w1a, w1b, w1c, b1, w2, b2, w3, b3):
  grid = (B // R,)
  full = lambda shape: pl.BlockSpec(shape, lambda i: (0, 0))
  return pl.pallas_call(
      _mlp_body,
      grid=grid,
      in_specs=[
          pl.BlockSpec((R, 10 * D), lambda i: (i, 0)),
          pl.BlockSpec((R, 3 * D), lambda i: (i, 0)),
          pl.BlockSpec((R, 8), lambda i: (i, 0)),
          full((10 * D, 64)),
          full((3 * D, 64)),
          full((8, 64)),
          full((1, 64)),
          full((64, 32)),
          full((1, 32)),
          full((32, 1)),
          full((1, 1)),
      ],
      out_specs=pl.BlockSpec((R, 1), lambda i: (i, 0)),
      out_shape=jax.ShapeDtypeStruct((B, 1), jnp.float32),
  )(g1, g2, info, w1a, w1b, w1c, b1, w2, b2, w3, b3)


def kernel(offensive_players, defensive_players, shooting_player,
           assisting_player, defending_player, is_putback, is_and1,
           is_freethrow, is_turnover, is_steal, shot_distance,
           emb_table, bag_table, W1, b1, W2, b2, W3, b3):
  bag_idx = jnp.concatenate(
      [offensive_players, defensive_players], axis=1
  ).astype(jnp.int32).reshape(NW, BAG_CH, IDX_CHUNK)
  emb_idx = jnp.stack(
      [shooting_player, assisting_player, defending_player], axis=1
  ).astype(jnp.int32).reshape(NW, EMB_CH, IDX_CHUNK)

  g1, g2 = _sc_gather(bag_idx, emb_idx,
                      bag_table.astype(jnp.float32),
                      emb_table.astype(jnp.float32))
  g1 = g1.reshape(B, 10 * D)
  g2 = g2.reshape(B, 3 * D)

  info = jnp.stack(
      [is_putback, is_and1, is_freethrow, is_turnover, is_steal,
       shot_distance], axis=1)
  info = jnp.pad(info, ((0, 0), (0, 2)))

  w1a = jnp.concatenate(
      [jnp.tile(W1[0:D], (5, 1)), jnp.tile(W1[D:2 * D], (5, 1))], axis=0)
  w1b = W1[2 * D:5 * D]
  w1c = jnp.pad(W1[5 * D:5 * D + 6], ((0, 2), (0, 0)))

  return _mlp(g1, g2, info, w1a, w1b, w1c,
              b1.reshape(1, 64), W2, b2.reshape(1, 32),
              W3, b3.reshape(1, 1))


# speedup vs baseline: 2.4908x; 2.4908x over previous
import functools

import jax
import jax.numpy as jnp
from jax import lax
from jax.experimental import pallas as pl
from jax.experimental.pallas import tpu as pltpu
from jax.experimental.pallas import tpu_sc as plsc

B = 16384
D = 64

NC = 2
NS = 16
NW = NC * NS

IDX_CHUNK = 128
BAG_CH = (B * 10) // (NW * IDX_CHUNK)
EMB_CH = (B * 3) // (NW * IDX_CHUNK)
GPO = 4


def _sc_gather_body(bag_idx, emb_idx, bag_table, emb_table,
                    out_bag, out_emb, idx_b_v, idx_e_v, rows_v, sem):
  wid = lax.axis_index("s") * NC + lax.axis_index("c")
  pltpu.sync_copy(bag_idx.at[wid], idx_b_v)
  pltpu.sync_copy(emb_idx.at[wid], idx_e_v)

  def bag_body(it, carry):
    cps = [
        pltpu.async_copy(bag_table.at[idx_b_v.at[it * GPO + g]],
                         rows_v.at[pl.ds(g * IDX_CHUNK, IDX_CHUNK)], sem)
        for g in range(GPO)
    ]
    for c in cps:
      c.wait()
    base = wid * (BAG_CH * IDX_CHUNK) + it * (GPO * IDX_CHUNK)
    pltpu.sync_copy(rows_v, out_bag.at[pl.ds(base, GPO * IDX_CHUNK)])
    return carry

  lax.fori_loop(0, BAG_CH // GPO, bag_body, 0, unroll=False)

  def emb_body(it, carry):
    cps = [
        pltpu.async_copy(emb_table.at[idx_e_v.at[it * GPO + g]],
                         rows_v.at[pl.ds(g * IDX_CHUNK, IDX_CHUNK)], sem)
        for g in range(GPO)
    ]
    for c in cps:
      c.wait()
    base = wid * (EMB_CH * IDX_CHUNK) + it * (GPO * IDX_CHUNK)
    pltpu.sync_copy(rows_v, out_emb.at[pl.ds(base, GPO * IDX_CHUNK)])
    return carry

  lax.fori_loop(0, EMB_CH // GPO, emb_body, 0, unroll=False)


def _sc_gather(bag_idx, emb_idx, bag_table, emb_table):
  mesh = plsc.VectorSubcoreMesh(core_axis_name="c", subcore_axis_name="s")
  fn = pl.kernel(
      _sc_gather_body,
      out_type=[
          jax.ShapeDtypeStruct((B * 10, D), jnp.float32),
          jax.ShapeDtypeStruct((B * 3, D), jnp.float32),
      ],
      mesh=mesh,
      scratch_types=[
          pltpu.VMEM((BAG_CH, IDX_CHUNK), jnp.int32),
          pltpu.VMEM((EMB_CH, IDX_CHUNK), jnp.int32),
          pltpu.VMEM((GPO * IDX_CHUNK, D), jnp.float32),
          pltpu.SemaphoreType.DMA,
      ],
      compiler_params=pltpu.CompilerParams(use_tc_tiling_on_sc=False),
  )
  return fn(bag_idx, emb_idx, bag_table, emb_table)


R = 1024


def _mlp_body(g1, g2, info, w1a, w1b, w1c, b1, w2, b2, w3, b3, out):
  h = (
      jnp.dot(g1[...], w1a[...], preferred_element_type=jnp.float32)
      + jnp.dot(g2[...], w1b[...], preferred_element_type=jnp.float32)
      + jnp.dot(info[...], w1c[...], preferred_element_type=jnp.float32)
      + b1[...]
  )
  h = jnp.maximum(h, 0.0)
  h = jnp.maximum(jnp.dot(h, w2[...], preferred_element_type=jnp.float32)
                  + b2[...], 0.0)
  out[...] = jnp.dot(h, w3[...], preferred_element_type=jnp.float32) + b3[...]


def _mlp(g1, g2, info, w1a, w1b, w1c, b1, w2, b2, w3, b3):
  grid = (B // R,)
  full = lambda shape: pl.BlockSpec(shape, lambda i: (0, 0))
  return pl.pallas_call(
      _mlp_body,
      grid=grid,
      in_specs=[
          pl.BlockSpec((R, 10 * D), lambda i: (i, 0)),
          pl.BlockSpec((R, 3 * D), lambda i: (i, 0)),
          pl.BlockSpec((R, 8), lambda i: (i, 0)),
          full((10 * D, 64)),
          full((3 * D, 64)),
          full((8, 64)),
          full((1, 64)),
          full((64, 32)),
          full((1, 32)),
          full((32, 1)),
          full((1, 1)),
      ],
      out_specs=pl.BlockSpec((R, 1), lambda i: (i, 0)),
      out_shape=jax.ShapeDtypeStruct((B, 1), jnp.float32),
  )(g1, g2, info, w1a, w1b, w1c, b1, w2, b2, w3, b3)


def kernel(offensive_players, defensive_players, shooting_player,
           assisting_player, defending_player, is_putback, is_and1,
           is_freethrow, is_turnover, is_steal, shot_distance,
           emb_table, bag_table, W1, b1, W2, b2, W3, b3):
  bag_idx = jnp.concatenate(
      [offensive_players, defensive_players], axis=1
  ).astype(jnp.int32).reshape(NW, BAG_CH, IDX_CHUNK)
  emb_idx = jnp.stack(
      [shooting_player, assisting_player, defending_player], axis=1
  ).astype(jnp.int32).reshape(NW, EMB_CH, IDX_CHUNK)

  g1, g2 = _sc_gather(bag_idx, emb_idx,
                      bag_table.astype(jnp.float32),
                      emb_table.astype(jnp.float32))
  g1 = g1.reshape(B, 10 * D)
  g2 = g2.reshape(B, 3 * D)

  info = jnp.stack(
      [is_putback, is_and1, is_freethrow, is_turnover, is_steal,
       shot_distance], axis=1)
  info = jnp.pad(info, ((0, 0), (0, 2)))

  w1a = jnp.concatenate(
      [jnp.tile(W1[0:D], (5, 1)), jnp.tile(W1[D:2 * D], (5, 1))], axis=0)
  w1b = W1[2 * D:5 * D]
  w1c = jnp.pad(W1[5 * D:5 * D + 6], ((0, 2), (0, 0)))

  return _mlp(g1, g2, info, w1a, w1b, w1c,
              b1.reshape(1, 64), W2, b2.reshape(1, 32),
              W3, b3.reshape(1, 1))


# speedup vs baseline: 2.8104x; 1.1283x over previous
import functools

import jax
import jax.numpy as jnp
from jax import lax
from jax.experimental import pallas as pl
from jax.experimental.pallas import tpu as pltpu
from jax.experimental.pallas import tpu_sc as plsc

B = 16384
V = 100000
D = 64

NC = 2
NS = 16
NW = NC * NS

VBLK = 1024
NVB = -(-V // VBLK)
VP = NVB * VBLK
BPW = B // NW
IDX_CHUNK = 128
NCH = BPW // IDX_CHUNK
NSLOT = 13


def _prep_body(bagT, embT, eye, out):
  t = (((0,), (0,)), ((), ()))
  out[:, 0:D] = lax.dot_general(bagT[...], eye[...], t,
                                preferred_element_type=jnp.float32)
  out[:, D:2 * D] = lax.dot_general(embT[...], eye[...], t,
                                    preferred_element_type=jnp.float32)


def _prep(bagT, embT):
  eye = jnp.eye(D, dtype=jnp.float32)
  return pl.pallas_call(
      _prep_body,
      grid=(NVB,),
      in_specs=[
          pl.BlockSpec((D, VBLK), lambda i: (0, i)),
          pl.BlockSpec((D, VBLK), lambda i: (0, i)),
          pl.BlockSpec((D, D), lambda i: (0, 0)),
      ],
      out_specs=pl.BlockSpec((VBLK, 2 * D), lambda i: (i, 0)),
      out_shape=jax.ShapeDtypeStruct((VP, 2 * D), jnp.float32),
  )(bagT, embT, eye)


def _sc_gather_body(fused, idx_flat, out, *scratch):
  idx_bufs = scratch[:NSLOT]
  rows_v, sem = scratch[NSLOT:]
  wid = lax.axis_index("s") * NC + lax.axis_index("c")
  base = wid * BPW
  for j in range(NSLOT):
    pltpu.sync_copy(idx_flat.at[pl.ds(j * B + base, BPW)], idx_bufs[j])
  for j in range(NSLOT):
    cps = [
        pltpu.async_copy(
            fused.at[idx_bufs[j].at[pl.ds(c * IDX_CHUNK, IDX_CHUNK)]],
            rows_v.at[pl.ds(c * IDX_CHUNK, IDX_CHUNK)], sem)
        for c in range(NCH)
    ]
    for cp in cps:
      cp.wait()
    pltpu.sync_copy(rows_v, out.at[j, pl.ds(base, BPW)])


def _sc_gather(fused, idx_flat):
  mesh = plsc.VectorSubcoreMesh(core_axis_name="c", subcore_axis_name="s")
  fn = pl.kernel(
      _sc_gather_body,
      out_type=jax.ShapeDtypeStruct((NSLOT, B, 2 * D), jnp.float32),
      mesh=mesh,
      scratch_types=(
          [pltpu.VMEM((BPW,), jnp.int32) for _ in range(NSLOT)]
          + [pltpu.VMEM((BPW, 2 * D), jnp.float32), pltpu.SemaphoreType.DMA]
      ),
      compiler_params=pltpu.CompilerParams(use_tc_tiling_on_sc=True),
  )
  return fn(fused, idx_flat)


R = 1024


def _mlp_body(slots, info, w1s, w1c, b1, w2, b2, w3, b3, out):
  h = jnp.dot(info[...], w1c[...], preferred_element_type=jnp.float32)
  for j in range(NSLOT):
    h = h + jnp.dot(slots[j], w1s[j], preferred_element_type=jnp.float32)
  h = jnp.maximum(h + b1[...], 0.0)
  h = jnp.maximum(jnp.dot(h, w2[...], preferred_element_type=jnp.float32)
                  + b2[...], 0.0)
  out[...] = jnp.dot(h, w3[...], preferred_element_type=jnp.float32) + b3[...]


def _mlp(slots, info, w1s, w1c, b1, w2, b2, w3, b3):
  full = lambda shape: pl.BlockSpec(shape, lambda i: (0,) * len(shape))
  return pl.pallas_call(
      _mlp_body,
      grid=(B // R,),
      in_specs=[
          pl.BlockSpec((NSLOT, R, 2 * D), lambda i: (0, i, 0)),
          pl.BlockSpec((R, 8), lambda i: (i, 0)),
          full((NSLOT, 2 * D, D)),
          full((8, D)),
          full((1, D)),
          full((D, 32)),
          full((1, 32)),
          full((32, 1)),
          full((1, 1)),
      ],
      out_specs=pl.BlockSpec((R, 1), lambda i: (i, 0)),
      out_shape=jax.ShapeDtypeStruct((B, 1), jnp.float32),
  )(slots, info, w1s, w1c, b1, w2, b2, w3, b3)


def kernel(offensive_players, defensive_players, shooting_player,
           assisting_player, defending_player, is_putback, is_and1,
           is_freethrow, is_turnover, is_steal, shot_distance,
           emb_table, bag_table, W1, b1, W2, b2, W3, b3):
  fused = _prep(bag_table.astype(jnp.float32).T,
                emb_table.astype(jnp.float32).T)

  idx_flat = jnp.concatenate([
      offensive_players.astype(jnp.int32).T.reshape(-1),
      defensive_players.astype(jnp.int32).T.reshape(-1),
      shooting_player.astype(jnp.int32),
      assisting_player.astype(jnp.int32),
      defending_player.astype(jnp.int32),
  ])
  slots = _sc_gather(fused, idx_flat)

  info = jnp.stack(
      [is_putback, is_and1, is_freethrow, is_turnover, is_steal,
       shot_distance], axis=1)
  info = jnp.pad(info, ((0, 0), (0, 2)))

  z = jnp.zeros((D, D), jnp.float32)
  bag_blk = lambda w: jnp.concatenate([w, z], axis=0)
  emb_blk = lambda w: jnp.concatenate([z, w], axis=0)
  w1s = jnp.stack([bag_blk(W1[0:D])] * 5 + [bag_blk(W1[D:2 * D])] * 5
                  + [emb_blk(W1[2 * D:3 * D]), emb_blk(W1[3 * D:4 * D]),
                     emb_blk(W1[4 * D:5 * D])])
  w1c = jnp.pad(W1[5 * D:5 * D + 6], ((0, 2), (0, 0)))

  return _mlp(slots, info, w1s, w1c,
              b1.reshape(1, D), W2, b2.reshape(1, 32),
              W3, b3.reshape(1, 1))


# speedup vs baseline: 3.1324x; 1.1146x over previous
import functools

import jax
import jax.numpy as jnp
from jax import lax
from jax.experimental import pallas as pl
from jax.experimental.pallas import tpu as pltpu
from jax.experimental.pallas import tpu_sc as plsc

B = 16384
V = 100000
D = 64

NC = 2
NS = 16
NW = NC * NS

VBLK = 2048
NVB = -(-V // VBLK)
VP = NVB * VBLK
BPW = B // NW
IDX_CHUNK = 128
NCH = BPW // IDX_CHUNK
NSLOT = 13


def _prep_body(bagT, embT, eye, out):
  t = (((0,), (0,)), ((), ()))
  out[:, 0:D] = lax.dot_general(bagT[...], eye[...], t,
                                preferred_element_type=jnp.float32)
  out[:, D:2 * D] = lax.dot_general(embT[...], eye[...], t,
                                    preferred_element_type=jnp.float32)


def _prep(bagT, embT):
  eye = jnp.eye(D, dtype=jnp.float32)
  return pl.pallas_call(
      _prep_body,
      grid=(NVB,),
      in_specs=[
          pl.BlockSpec((D, VBLK), lambda i: (0, i)),
          pl.BlockSpec((D, VBLK), lambda i: (0, i)),
          pl.BlockSpec((D, D), lambda i: (0, 0)),
      ],
      out_specs=pl.BlockSpec((VBLK, 2 * D), lambda i: (i, 0)),
      out_shape=jax.ShapeDtypeStruct((VP, 2 * D), jnp.float32),
      compiler_params=pltpu.CompilerParams(
          fuse_transposed_lhs_in_matmul=True),
  )(bagT, embT, eye)


def _sc_gather_body(fused, idx_flat, out, *scratch):
  idx_bufs = scratch[:NSLOT]
  rows_v, sem = scratch[NSLOT:]
  wid = lax.axis_index("s") * NC + lax.axis_index("c")
  base = wid * BPW
  for j in range(NSLOT):
    pltpu.sync_copy(idx_flat.at[pl.ds(j * B + base, BPW)], idx_bufs[j])
  for j in range(NSLOT):
    cps = [
        pltpu.async_copy(
            fused.at[idx_bufs[j].at[pl.ds(c * IDX_CHUNK, IDX_CHUNK)]],
            rows_v.at[pl.ds(c * IDX_CHUNK, IDX_CHUNK)], sem)
        for c in range(NCH)
    ]
    for cp in cps:
      cp.wait()
    pltpu.sync_copy(rows_v, out.at[j, pl.ds(base, BPW)])


def _sc_gather(fused, idx_flat):
  mesh = plsc.VectorSubcoreMesh(core_axis_name="c", subcore_axis_name="s")
  fn = pl.kernel(
      _sc_gather_body,
      out_type=jax.ShapeDtypeStruct((NSLOT, B, 2 * D), jnp.float32),
      mesh=mesh,
      scratch_types=(
          [pltpu.VMEM((BPW,), jnp.int32) for _ in range(NSLOT)]
          + [pltpu.VMEM((BPW, 2 * D), jnp.float32), pltpu.SemaphoreType.DMA]
      ),
      compiler_params=pltpu.CompilerParams(use_tc_tiling_on_sc=True),
  )
  return fn(fused, idx_flat)


R = 1024


def _mlp_body(slots, info, w1s, w1c, b1, w2, b2, w3, b3, out):
  h = jnp.dot(info[...], w1c[...], preferred_element_type=jnp.float32)
  for j in range(NSLOT):
    h = h + jnp.dot(slots[j], w1s[j], preferred_element_type=jnp.float32)
  h = jnp.maximum(h + b1[...], 0.0)
  h = jnp.maximum(jnp.dot(h, w2[...], preferred_element_type=jnp.float32)
                  + b2[...], 0.0)
  out[...] = jnp.dot(h, w3[...], preferred_element_type=jnp.float32) + b3[...]


def _mlp(slots, info, w1s, w1c, b1, w2, b2, w3, b3):
  full = lambda shape: pl.BlockSpec(shape, lambda i: (0,) * len(shape))
  return pl.pallas_call(
      _mlp_body,
      grid=(B // R,),
      in_specs=[
          pl.BlockSpec((NSLOT, R, 2 * D), lambda i: (0, i, 0)),
          pl.BlockSpec((R, 8), lambda i: (i, 0)),
          full((NSLOT, 2 * D, D)),
          full((8, D)),
          full((1, D)),
          full((D, 32)),
          full((1, 32)),
          full((32, 1)),
          full((1, 1)),
      ],
      out_specs=pl.BlockSpec((R, 1), lambda i: (i, 0)),
      out_shape=jax.ShapeDtypeStruct((B, 1), jnp.float32),
  )(slots, info, w1s, w1c, b1, w2, b2, w3, b3)


def kernel(offensive_players, defensive_players, shooting_player,
           assisting_player, defending_player, is_putback, is_and1,
           is_freethrow, is_turnover, is_steal, shot_distance,
           emb_table, bag_table, W1, b1, W2, b2, W3, b3):
  fused = _prep(bag_table.astype(jnp.float32).T,
                emb_table.astype(jnp.float32).T)

  idx_flat = jnp.concatenate([
      offensive_players.astype(jnp.int32).T.reshape(-1),
      defensive_players.astype(jnp.int32).T.reshape(-1),
      shooting_player.astype(jnp.int32),
      assisting_player.astype(jnp.int32),
      defending_player.astype(jnp.int32),
  ])
  slots = _sc_gather(fused, idx_flat)

  info = jnp.stack(
      [is_putback, is_and1, is_freethrow, is_turnover, is_steal,
       shot_distance], axis=1)
  info = jnp.pad(info, ((0, 0), (0, 2)))

  z = jnp.zeros((D, D), jnp.float32)
  bag_blk = lambda w: jnp.concatenate([w, z], axis=0)
  emb_blk = lambda w: jnp.concatenate([z, w], axis=0)
  w1s = jnp.stack([bag_blk(W1[0:D])] * 5 + [bag_blk(W1[D:2 * D])] * 5
                  + [emb_blk(W1[2 * D:3 * D]), emb_blk(W1[3 * D:4 * D]),
                     emb_blk(W1[4 * D:5 * D])])
  w1c = jnp.pad(W1[5 * D:5 * D + 6], ((0, 2), (0, 0)))

  return _mlp(slots, info, w1s, w1c,
              b1.reshape(1, D), W2, b2.reshape(1, 32),
              W3, b3.reshape(1, 1))
